# Initial kernel scaffold; baseline (speedup 1.0000x reference)
import jax
import jax.numpy as jnp
from jax.experimental import pallas as pl


def kernel(x):
    raise NotImplementedError("write your pallas kernel here")


# speedup vs baseline: 2.8705x; 2.8705x over previous
import functools

import jax
import jax.numpy as jnp
from jax import lax
from jax.experimental import pallas as pl
from jax.experimental.pallas import tpu as pltpu
from jax.experimental.pallas import tpu_sc as plsc

_K = 64
_B = 32
_N = 10000
_D = 128
_NPAD = 10240
_BIG = 1 << 30


def _topk_body(keys_ref, out_ref):
    keys = keys_ref[...]
    iota = lax.broadcasted_iota(jnp.int32, (_B, _NPAD), 1)
    col = lax.broadcasted_iota(jnp.int32, (_B, _K), 1)

    def step(j, carry):
        keys, acc = carry
        m = jnp.max(keys, axis=1, keepdims=True)
        sel = jnp.min(jnp.where(keys == m, iota, jnp.int32(_BIG)),
                      axis=1, keepdims=True)
        keys = jnp.where(iota == sel, -jnp.inf, keys)
        acc = jnp.where(col == j, sel, acc)
        return keys, acc

    _, acc = lax.fori_loop(
        0, _K, step, (keys, jnp.zeros((_B, _K), jnp.int32)))
    bid = lax.broadcasted_iota(jnp.int32, (_B, _K), 0)
    out_ref[...] = acc + bid * _N


def _topk_indices(keys_padded):
    return pl.pallas_call(
        _topk_body,
        out_shape=jax.ShapeDtypeStruct((_B, _K), jnp.int32),
    )(keys_padded)


@functools.lru_cache(maxsize=1)
def _make_gather():
    info = plsc.get_sparse_core_info()
    nw = info.num_cores * info.num_subcores
    rpw = (_B * _K) // nw
    mesh = plsc.VectorSubcoreMesh(core_axis_name="c", subcore_axis_name="s")

    @functools.partial(
        pl.kernel,
        mesh=mesh,
        out_type=jax.ShapeDtypeStruct((_B * _K, _D), jnp.float32),
        scratch_types=[
            pltpu.VMEM((rpw,), jnp.int32),
            pltpu.VMEM((rpw, _D), jnp.float32),
            pltpu.SemaphoreType.DMA,
        ],
    )
    def gather(table_hbm, idx_hbm, out_hbm, idx_v, rows_v, sem):
        wid = lax.axis_index("s") * info.num_cores + lax.axis_index("c")
        base = wid * rpw
        pltpu.sync_copy(idx_hbm.at[pl.ds(base, rpw)], idx_v)
        pltpu.async_copy(table_hbm.at[idx_v], rows_v, sem).wait()
        pltpu.sync_copy(rows_v, out_hbm.at[pl.ds(base, rpw)])

    return gather


def kernel(x):
    keys = x[:, :, _D - 1]
    keys = jnp.pad(keys, ((0, 0), (0, _NPAD - _N)),
                   constant_values=-jnp.inf)
    flat_idx = _topk_indices(keys).reshape(_B * _K)
    table = x.reshape(_B * _N, _D)
    out = _make_gather()(table, flat_idx)
    return out.reshape(_B, _K * _D)


# speedup vs baseline: 7.4703x; 2.6024x over previous
import functools

import jax
import jax.numpy as jnp
from jax import lax
from jax.experimental import pallas as pl
from jax.experimental.pallas import tpu as pltpu
from jax.experimental.pallas import tpu_sc as plsc

_K = 64
_B = 32
_N = 10000
_D = 128
_NPAD = 10240
_BIG = 1 << 30
_CHUNK = 128
_GROUP = 8
_NGRP = _NPAD // (_CHUNK * _GROUP)


def _topk_body(keys_ref, out_ref):
    keys = keys_ref[...]
    iota = lax.broadcasted_iota(jnp.int32, (_B, _NPAD), 1)
    col = lax.broadcasted_iota(jnp.int32, (_B, _K), 1)

    def step(j, carry):
        keys, acc = carry
        m = jnp.max(keys, axis=1, keepdims=True)
        sel = jnp.min(jnp.where(keys == m, iota, jnp.int32(_BIG)),
                      axis=1, keepdims=True)
        keys = jnp.where(iota == sel, -jnp.inf, keys)
        acc = jnp.where(col == j, sel, acc)
        return keys, acc

    _, acc = lax.fori_loop(
        0, _K, step, (keys, jnp.zeros((_B, _K), jnp.int32)))
    bid = lax.broadcasted_iota(jnp.int32, (_B, _K), 0)
    out_ref[...] = acc + bid * _N


def _topk_indices(keys_padded):
    return pl.pallas_call(
        _topk_body,
        out_shape=jax.ShapeDtypeStruct((_B, _K), jnp.int32),
    )(keys_padded)


@functools.lru_cache(maxsize=1)
def _make_keys_extract():
    info = plsc.get_sparse_core_info()
    nc = info.num_cores
    mesh = plsc.VectorSubcoreMesh(core_axis_name="c", subcore_axis_name="s")

    @functools.partial(
        pl.kernel,
        mesh=mesh,
        out_type=jax.ShapeDtypeStruct((_B, _NPAD), jnp.float32),
        scratch_types=[
            pltpu.VMEM((_NPAD,), jnp.int32),
            pltpu.VMEM((_NPAD,), jnp.float32),
            pltpu.SemaphoreType.DMA,
        ],
    )
    def keys_extract(xflat_hbm, out_hbm, idx_v, keys_v, sem):
        b = lax.axis_index("s") * nc + lax.axis_index("c")
        base = b * _N
        iota = lax.broadcasted_iota(jnp.int32, (16,), 0)

        def build(j, carry):
            i = jnp.minimum(j * 16 + iota, _N - 1)
            idx_v[pl.ds(j * 16, 16)] = (base + i) * _D + (_D - 1)
            return carry

        lax.fori_loop(0, _NPAD // 16, build, 0)

        def group(g, carry):
            cbase = g * _GROUP
            copies = []
            for p in range(_GROUP):
                off = (cbase + p) * _CHUNK
                copies.append(pltpu.async_copy(
                    xflat_hbm.at[idx_v.at[pl.ds(off, _CHUNK)]],
                    keys_v.at[pl.ds(off, _CHUNK)], sem))
            for cp in copies:
                cp.wait()
            return carry

        lax.fori_loop(0, _NGRP, group, 0)

        def tail(t, carry):
            keys_v[pl.ds(_N + t * 16, 16)] = jnp.full(
                (16,), -jnp.inf, jnp.float32)
            return carry

        lax.fori_loop(0, (_NPAD - _N) // 16, tail, 0)
        pltpu.sync_copy(keys_v, out_hbm.at[b])

    return keys_extract


@functools.lru_cache(maxsize=1)
def _make_gather():
    info = plsc.get_sparse_core_info()
    nw = info.num_cores * info.num_subcores
    rpw = (_B * _K) // nw
    mesh = plsc.VectorSubcoreMesh(core_axis_name="c", subcore_axis_name="s")

    @functools.partial(
        pl.kernel,
        mesh=mesh,
        out_type=jax.ShapeDtypeStruct((_B * _K, _D), jnp.float32),
        scratch_types=[
            pltpu.VMEM((rpw,), jnp.int32),
            pltpu.VMEM((rpw, _D), jnp.float32),
            pltpu.SemaphoreType.DMA,
        ],
    )
    def gather(table_hbm, idx_hbm, out_hbm, idx_v, rows_v, sem):
        wid = lax.axis_index("s") * info.num_cores + lax.axis_index("c")
        base = wid * rpw
        pltpu.sync_copy(idx_hbm.at[pl.ds(base, rpw)], idx_v)
        pltpu.async_copy(table_hbm.at[idx_v], rows_v, sem).wait()
        pltpu.sync_copy(rows_v, out_hbm.at[pl.ds(base, rpw)])

    return gather


def kernel(x):
    xflat = x.reshape(_B * _N * _D)
    keys = _make_keys_extract()(xflat)
    flat_idx = _topk_indices(keys).reshape(_B * _K)
    table = x.reshape(_B * _N, _D)
    out = _make_gather()(table, flat_idx)
    return out.reshape(_B, _K * _D)


# speedup vs baseline: 9.6123x; 1.2867x over previous
import functools

import jax
import jax.numpy as jnp
from jax import lax
from jax.experimental import pallas as pl
from jax.experimental.pallas import tpu as pltpu
from jax.experimental.pallas import tpu_sc as plsc

_K = 64
_B = 32
_N = 10000
_D = 128
_NPAD = 10240
_CHUNK = 128
_NCHUNK = _NPAD // _CHUNK
_MININT = -(2 ** 31)


def _mesh():
    return plsc.VectorSubcoreMesh(core_axis_name="c", subcore_axis_name="s")


def _wid(nc):
    return lax.axis_index("s") * nc + lax.axis_index("c")


def _s32(v):
    b = lax.bitcast_convert_type(v, jnp.int32)
    return jnp.where(b < 0, b ^ jnp.int32(0x7FFFFFFF), b)


@functools.lru_cache(maxsize=1)
def _make_keys_extract():
    info = plsc.get_sparse_core_info()
    nc = info.num_cores

    @functools.partial(
        pl.kernel,
        mesh=_mesh(),
        out_type=jax.ShapeDtypeStruct((_B, _NPAD), jnp.float32),
        scratch_types=[
            pltpu.VMEM((_NPAD,), jnp.int32),
            pltpu.VMEM((_NPAD,), jnp.float32),
            pltpu.SemaphoreType.DMA,
        ],
    )
    def keys_extract(xflat_hbm, out_hbm, idx_v, keys_v, sem):
        b = _wid(nc)
        base = b * _N
        iota = lax.broadcasted_iota(jnp.int32, (16,), 0)

        def build(j, carry):
            i = jnp.minimum(j * 16 + iota, _N - 1)
            idx_v[pl.ds(j * 16, 16)] = (base + i) * _D + (_D - 1)
            return carry

        lax.fori_loop(0, _NPAD // 16, build, 0)

        def fire(c, carry):
            off = c * _CHUNK
            pltpu.async_copy(
                xflat_hbm.at[idx_v.at[pl.ds(off, _CHUNK)]],
                keys_v.at[pl.ds(off, _CHUNK)], sem)
            return carry

        lax.fori_loop(0, _NCHUNK, fire, 0)

        def drain(c, carry):
            off = c * _CHUNK
            pltpu.make_async_copy(
                xflat_hbm.at[idx_v.at[pl.ds(off, _CHUNK)]],
                keys_v.at[pl.ds(off, _CHUNK)], sem).wait()
            return carry

        lax.fori_loop(0, _NCHUNK, drain, 0)

        def tail(t, carry):
            keys_v[pl.ds(_N + t * 16, 16)] = jnp.full(
                (16,), -jnp.inf, jnp.float32)
            return carry

        lax.fori_loop(0, (_NPAD - _N) // 16, tail, 0)
        pltpu.sync_copy(keys_v, out_hbm.at[b])

    return keys_extract


def _thresh_body(keys_ref, out_ref, pgt_ref, peq_ref):
    s = _s32(keys_ref[...])

    def step(j, t_u):
        bit = lax.shift_left(jnp.int32(1), 31 - j)
        try_u = t_u | bit
        cnt = jnp.sum((s >= (try_u ^ jnp.int32(_MININT))).astype(jnp.int32),
                      axis=1, keepdims=True)
        return jnp.where(cnt >= _K, try_u, t_u)

    t_u = lax.fori_loop(0, 32, step, jnp.zeros((_B, 1), jnp.int32))
    t_s = t_u ^ jnp.int32(_MININT)
    m_gt = s > t_s
    m_eq = s == t_s
    cnt_gt = jnp.sum(m_gt.astype(jnp.int32), axis=1, keepdims=True)
    lane = lax.broadcasted_iota(jnp.int32, (_B, 128), 1)
    out = jnp.where(lane == 0, t_s, jnp.int32(0))
    out = jnp.where(lane == 1, _K - cnt_gt, out)
    out_ref[...] = out
    c = jnp.where(m_gt, jnp.int32(1 << 16), jnp.int32(0)) + \
        jnp.where(m_eq, jnp.int32(1), jnp.int32(0))
    d = 1
    while d < _NPAD:
        c = c + jnp.concatenate(
            [jnp.zeros((_B, d), jnp.int32), c[:, :-d]], axis=1)
        d *= 2
    pgt_ref[...] = lax.shift_right_logical(c, 16)
    peq_ref[...] = c & jnp.int32(0xFFFF)


def _threshold(keys):
    return pl.pallas_call(
        _thresh_body,
        out_shape=[
            jax.ShapeDtypeStruct((_B, 128), jnp.int32),
            jax.ShapeDtypeStruct((_B, _NPAD), jnp.int32),
            jax.ShapeDtypeStruct((_B, _NPAD), jnp.int32),
        ],
    )(keys)


@functools.lru_cache(maxsize=1)
def _make_select_gather():
    info = plsc.get_sparse_core_info()
    nc = info.num_cores

    @functools.partial(
        pl.kernel,
        mesh=_mesh(),
        out_type=jax.ShapeDtypeStruct((_B, _K, _D), jnp.float32),
        scratch_types=[
            pltpu.VMEM((_NPAD,), jnp.float32),
            pltpu.VMEM((_NPAD,), jnp.int32),
            pltpu.VMEM((_NPAD,), jnp.int32),
            pltpu.VMEM((128,), jnp.int32),
            pltpu.VMEM((_K + 96,), jnp.int32),
            pltpu.VMEM((_K + 96,), jnp.int32),
            pltpu.VMEM((_K + 96,), jnp.int32),
            pltpu.VMEM((_K,), jnp.int32),
            pltpu.VMEM((_K, _D), jnp.float32),
            pltpu.SMEM((_K,), jnp.int32),
            pltpu.SemaphoreType.DMA,
        ],
    )
    def select_gather(keys_hbm, thr_hbm, pgt_hbm, peq_hbm, table_hbm,
                      out_hbm, keys_v, pgt_v, peq_v, thr_v, gts_v, gti_v,
                      eqi_v, ord_v, rows_v, ord_sm, sem):
        b = _wid(nc)
        iota = lax.broadcasted_iota(jnp.int32, (16,), 0)
        pltpu.sync_copy(keys_hbm.at[b], keys_v)
        pltpu.sync_copy(pgt_hbm.at[b], pgt_v)
        pltpu.sync_copy(peq_hbm.at[b], peq_v)
        pltpu.sync_copy(thr_hbm.at[b], thr_v)
        tv = thr_v[pl.ds(0, 16)]
        t_s = tv[0]
        need_eq = tv[1]
        cnt_gt = _K - need_eq

        trash = _K + 80
        ones16 = jnp.full((16,), 1, jnp.int32)

        def scan(j, carry):
            pg_prev, pe_prev = carry
            pgv = pgt_v[pl.ds(j * 16, 16)]
            pev = peq_v[pl.ds(j * 16, 16)]
            pg15 = pgv[15]
            pe15 = pev[15]

            @pl.when(pg15 - pg_prev + pe15 - pe_prev > 0)
            def _():
                s = _s32(keys_v[pl.ds(j * 16, 16)])
                for l in range(16):
                    sl = s[l]
                    sl16 = ones16 * sl
                    il16 = ones16 * (j * 16 + l)
                    is_gt = sl > t_s
                    is_eq = sl == t_s
                    dst = jnp.where(is_gt, pgv[l] - 1, trash)
                    gts_v[pl.ds(dst, 16)] = sl16
                    gti_v[pl.ds(dst, 16)] = il16
                    edst = jnp.where(is_eq & (pev[l] - 1 < need_eq),
                                     pev[l] - 1, trash)
                    eqi_v[pl.ds(edst, 16)] = il16

            return pg15, pe15

        lax.fori_loop(0, _NPAD // 16, scan, (jnp.int32(0), jnp.int32(0)))

        for v in range(_K // 16):
            gts_v[pl.ds(cnt_gt + v * 16, 16)] = ones16 * t_s
            gti_v[pl.ds(cnt_gt + v * 16, 16)] = eqi_v[pl.ds(v * 16, 16)]

        ranks = [jnp.zeros((16,), jnp.int32) for _ in range(_K // 16)]
        svecs = [gts_v[pl.ds(v * 16, 16)] for v in range(_K // 16)]
        ivecs = [gti_v[pl.ds(v * 16, 16)] for v in range(_K // 16)]

        def rank_step(j, rs):
            sj = gts_v[pl.ds(j, 16)][0]
            ij = gti_v[pl.ds(j, 16)][0]
            out = []
            for v in range(_K // 16):
                beat = (sj > svecs[v]) | ((sj == svecs[v]) & (ij < ivecs[v]))
                out.append(rs[v] + jnp.where(beat, 1, 0))
            return tuple(out)

        ranks = lax.fori_loop(0, _K, rank_step, tuple(ranks))

        for v in range(_K // 16):
            for l in range(16):
                ord_sm[ranks[v][l]] = ivecs[v][l]
        for v in range(_K // 16):
            vec = jnp.zeros((16,), jnp.int32)
            for l in range(16):
                vec = jnp.where(iota == l, ord_sm[v * 16 + l], vec)
            ord_v[pl.ds(v * 16, 16)] = b * _N + vec

        pltpu.async_copy(table_hbm.at[ord_v], rows_v, sem).wait()
        pltpu.sync_copy(rows_v, out_hbm.at[b])

    return select_gather


def kernel(x):
    xflat = x.reshape(_B * _N * _D)
    keys = _make_keys_extract()(xflat)
    thr, pgt, peq = _threshold(keys)
    table = x.reshape(_B * _N, _D)
    out = _make_select_gather()(keys, thr, pgt, peq, table)
    return out.reshape(_B, _K * _D)


# speedup vs baseline: 10.2779x; 1.0692x over previous
import functools

import jax
import jax.numpy as jnp
from jax import lax
from jax.experimental import pallas as pl
from jax.experimental.pallas import tpu as pltpu
from jax.experimental.pallas import tpu_sc as plsc

_K = 64
_B = 32
_N = 10000
_D = 128
_NPAD = 10240
_CHUNK = 128
_NCHUNK = _NPAD // _CHUNK
_MININT = -(2 ** 31)


def _mesh():
    return plsc.VectorSubcoreMesh(core_axis_name="c", subcore_axis_name="s")


def _wid(nc):
    return lax.axis_index("s") * nc + lax.axis_index("c")


def _s32(v):
    b = lax.bitcast_convert_type(v, jnp.int32)
    return jnp.where(b < 0, b ^ jnp.int32(0x7FFFFFFF), b)


@functools.lru_cache(maxsize=1)
def _make_keys_extract():
    info = plsc.get_sparse_core_info()
    nc = info.num_cores

    @functools.partial(
        pl.kernel,
        mesh=_mesh(),
        out_type=jax.ShapeDtypeStruct((_B, _NPAD), jnp.float32),
        scratch_types=[
            pltpu.VMEM((_NPAD,), jnp.int32),
            pltpu.VMEM((_NPAD,), jnp.float32),
            pltpu.SemaphoreType.DMA,
        ],
    )
    def keys_extract(xflat_hbm, idx_hbm, out_hbm, idx_v, keys_v, sem):
        b = _wid(nc)
        pltpu.sync_copy(idx_hbm.at[b], idx_v)

        def fire(c, carry):
            off = c * _CHUNK
            pltpu.async_copy(
                xflat_hbm.at[idx_v.at[pl.ds(off, _CHUNK)]],
                keys_v.at[pl.ds(off, _CHUNK)], sem)
            return carry

        lax.fori_loop(0, _NCHUNK, fire, 0)

        def drain(c, carry):
            off = c * _CHUNK
            pltpu.make_async_copy(
                xflat_hbm.at[idx_v.at[pl.ds(off, _CHUNK)]],
                keys_v.at[pl.ds(off, _CHUNK)], sem).wait()
            return carry

        lax.fori_loop(0, _NCHUNK, drain, 0)

        def tail(t, carry):
            keys_v[pl.ds(_N + t * 16, 16)] = jnp.full(
                (16,), -jnp.inf, jnp.float32)
            return carry

        lax.fori_loop(0, (_NPAD - _N) // 16, tail, 0)
        pltpu.sync_copy(keys_v, out_hbm.at[b])

    return keys_extract


def _thresh_body(keys_ref, out_ref, hint_ref):
    s = _s32(keys_ref[...])

    def step(j, t_u):
        bit = lax.shift_left(jnp.int32(1), 31 - j)
        try_u = t_u | bit
        cnt = jnp.sum((s >= (try_u ^ jnp.int32(_MININT))).astype(jnp.int32),
                      axis=1, keepdims=True)
        return jnp.where(cnt >= _K, try_u, t_u)

    t_u = lax.fori_loop(0, 32, step, jnp.zeros((_B, 1), jnp.int32))
    t_s = t_u ^ jnp.int32(_MININT)
    m_ge = s >= t_s
    cnt_gt = jnp.sum((s > t_s).astype(jnp.int32), axis=1, keepdims=True)
    lane = lax.broadcasted_iota(jnp.int32, (_B, 128), 1)
    out = jnp.where(lane == 0, t_s, jnp.int32(0))
    out = jnp.where(lane == 1, _K - cnt_gt, out)
    out_ref[...] = out
    w = m_ge.astype(jnp.int32)
    for d in (1, 2, 4, 8):
        w = w + jnp.concatenate(
            [jnp.zeros((_B, d), jnp.int32), w[:, :-d]], axis=1)
    hint_ref[...] = w


def _threshold(keys):
    return pl.pallas_call(
        _thresh_body,
        out_shape=[
            jax.ShapeDtypeStruct((_B, 128), jnp.int32),
            jax.ShapeDtypeStruct((_B, _NPAD), jnp.int32),
        ],
    )(keys)


@functools.lru_cache(maxsize=1)
def _make_select_gather():
    info = plsc.get_sparse_core_info()
    nc = info.num_cores

    @functools.partial(
        pl.kernel,
        mesh=_mesh(),
        out_type=jax.ShapeDtypeStruct((_B, _K, _D), jnp.float32),
        scratch_types=[
            pltpu.VMEM((_NPAD,), jnp.float32),
            pltpu.VMEM((_NPAD,), jnp.int32),
            pltpu.VMEM((128,), jnp.int32),
            pltpu.VMEM((_K + 96,), jnp.int32),
            pltpu.VMEM((_K + 96,), jnp.int32),
            pltpu.VMEM((_K + 96,), jnp.int32),
            pltpu.VMEM((_K,), jnp.int32),
            pltpu.VMEM((_K, _D), jnp.float32),
            pltpu.SMEM((_K,), jnp.int32),
            pltpu.SMEM((4,), jnp.int32),
            pltpu.SemaphoreType.DMA,
        ],
    )
    def select_gather(keys_hbm, thr_hbm, hint_hbm, table_hbm,
                      out_hbm, keys_v, hint_v, thr_v, gts_v, gti_v,
                      eqi_v, ord_v, rows_v, ord_sm, cnt_sm, sem):
        b = _wid(nc)
        iota = lax.broadcasted_iota(jnp.int32, (16,), 0)
        cp_k = pltpu.async_copy(keys_hbm.at[b], keys_v, sem)
        cp_h = pltpu.async_copy(hint_hbm.at[b], hint_v, sem)
        cp_t = pltpu.async_copy(thr_hbm.at[b], thr_v, sem)
        cp_k.wait()
        cp_h.wait()
        cp_t.wait()
        tv = thr_v[pl.ds(0, 16)]
        t_s = tv[0]
        need_eq = tv[1]
        cnt_gt = _K - need_eq

        trash = _K + 80
        ones16 = jnp.full((16,), 1, jnp.int32)
        cnt_sm[0] = 0
        cnt_sm[1] = 0

        def scan(j, carry):
            hv = hint_v[pl.ds(j * 16, 16)]

            @pl.when(hv[15] > 0)
            def _():
                s = _s32(keys_v[pl.ds(j * 16, 16)])
                p = cnt_sm[0]
                q = cnt_sm[1]
                for l in range(16):
                    sl = s[l]
                    il16 = ones16 * (j * 16 + l)
                    is_gt = sl > t_s
                    is_eq = sl == t_s
                    dst = jnp.where(is_gt, p, trash)
                    gts_v[pl.ds(dst, 16)] = ones16 * sl
                    gti_v[pl.ds(dst, 16)] = il16
                    edst = jnp.where(is_eq & (q < need_eq), q, trash)
                    eqi_v[pl.ds(edst, 16)] = il16
                    p = p + jnp.where(is_gt, 1, 0)
                    q = q + jnp.where(is_eq, 1, 0)
                cnt_sm[0] = p
                cnt_sm[1] = q

            return carry

        lax.fori_loop(0, _NPAD // 16, scan, 0)

        for v in range(_K // 16):
            gts_v[pl.ds(cnt_gt + v * 16, 16)] = ones16 * t_s
            gti_v[pl.ds(cnt_gt + v * 16, 16)] = eqi_v[pl.ds(v * 16, 16)]

        ranks = [jnp.zeros((16,), jnp.int32) for _ in range(_K // 16)]
        svecs = [gts_v[pl.ds(v * 16, 16)] for v in range(_K // 16)]
        ivecs = [gti_v[pl.ds(v * 16, 16)] for v in range(_K // 16)]

        def rank_step(j, rs):
            sj = gts_v[pl.ds(j, 16)][0]
            ij = gti_v[pl.ds(j, 16)][0]
            out = []
            for v in range(_K // 16):
                beat = (sj > svecs[v]) | ((sj == svecs[v]) & (ij < ivecs[v]))
                out.append(rs[v] + jnp.where(beat, 1, 0))
            return tuple(out)

        ranks = lax.fori_loop(0, _K, rank_step, tuple(ranks))

        for v in range(_K // 16):
            for l in range(16):
                ord_sm[ranks[v][l]] = ivecs[v][l]
        for v in range(_K // 16):
            vec = jnp.zeros((16,), jnp.int32)
            for l in range(16):
                vec = jnp.where(iota == l, ord_sm[v * 16 + l], vec)
            ord_v[pl.ds(v * 16, 16)] = b * _N + vec

        pltpu.async_copy(table_hbm.at[ord_v], rows_v, sem).wait()
        pltpu.sync_copy(rows_v, out_hbm.at[b])

    return select_gather


def kernel(x):
    xflat = x.reshape(_B * _N * _D)
    node = jnp.minimum(jax.lax.iota(jnp.int32, _NPAD), _N - 1)
    gidx = ((jax.lax.iota(jnp.int32, _B)[:, None] * _N + node[None, :])
            * _D + (_D - 1))
    keys = _make_keys_extract()(xflat, gidx)
    thr, hint = _threshold(keys)
    table = x.reshape(_B * _N, _D)
    out = _make_select_gather()(keys, thr, hint, table)
    return out.reshape(_B, _K * _D)
